# baseline (speedup 1.0000x reference)
import functools

import jax
import jax.numpy as jnp
from jax import lax
from jax.experimental import pallas as pl
from jax.experimental.pallas import tpu as pltpu
from jax.experimental.pallas import tpu_sc as plsc

NN = 10000
DG = 16
DD = 256
NW = 32
NPAD = 10240
PER_W = NPAD // NW
CHUNK = 8
NCHUNK = PER_W // CHUNK
NBUF = 3
LANES = 16


def _sc_gather_sum(table, idx):
    mesh = plsc.VectorSubcoreMesh(core_axis_name="c", subcore_axis_name="s")

    @functools.partial(
        pl.kernel,
        mesh=mesh,
        out_type=jax.ShapeDtypeStruct((NPAD, DD), jnp.float32),
        scratch_types=(
            [pltpu.VMEM((PER_W * DG,), jnp.int32),
             pltpu.VMEM((NBUF, CHUNK * DG, DD), jnp.float32),
             pltpu.VMEM((NBUF, CHUNK, DD), jnp.float32)]
            + [pltpu.SemaphoreType.DMA] * (2 * NBUF)
        ),
    )
    def k(table_hbm, idx_hbm, out_hbm, idx_v, rows_v, acc_v, *sems):
        wid = lax.axis_index("s") * 2 + lax.axis_index("c")
        base = wid * PER_W
        pltpu.sync_copy(idx_hbm.at[pl.ds(base * DG, PER_W * DG)], idx_v)

        gsems = sems[:NBUF]
        osems = sems[NBUF:]

        def issue_gather(c, slot):
            return pltpu.async_copy(
                table_hbm.at[idx_v.at[pl.ds(c * CHUNK * DG, CHUNK * DG)]],
                rows_v.at[slot], gsems[slot])

        for slot in range(NBUF):
            issue_gather(slot, slot)

        def wait_gather(slot):
            pltpu.make_async_copy(
                table_hbm.at[pl.ds(0, CHUNK * DG)],
                rows_v.at[slot], gsems[slot]).wait()

        def reduce_slab(slot):
            def node_body(n, _):
                r0 = n * DG
                for d in range(DD // LANES):
                    sl = pl.ds(d * LANES, LANES)
                    vals = [rows_v[slot, r0 + j, sl] for j in range(DG)]
                    while len(vals) > 1:
                        vals = [vals[i] + vals[i + 1]
                                for i in range(0, len(vals), 2)]
                    acc_v[slot, n, sl] = vals[0]
                return 0
            lax.fori_loop(0, CHUNK, node_body, 0)

        def group_body(t, _):
            for slot in range(NBUF):
                c = NBUF * t + slot
                wait_gather(slot)

                @pl.when(t > 0)
                def _():
                    pltpu.make_async_copy(
                        acc_v.at[slot],
                        out_hbm.at[pl.ds(base, CHUNK)], osems[slot]).wait()

                reduce_slab(slot)

                @pl.when(c + NBUF < NCHUNK)
                def _():
                    issue_gather(c + NBUF, slot)

                pltpu.async_copy(
                    acc_v.at[slot],
                    out_hbm.at[pl.ds(base + c * CHUNK, CHUNK)], osems[slot])
            return 0

        ngroups = NCHUNK // NBUF
        lax.fori_loop(0, ngroups, group_body, 0)
        for r in range(NCHUNK % NBUF):
            c = ngroups * NBUF + r
            wait_gather(r)
            pltpu.make_async_copy(
                acc_v.at[r], out_hbm.at[pl.ds(base, CHUNK)],
                osems[r]).wait()
            reduce_slab(r)
            pltpu.async_copy(
                acc_v.at[r],
                out_hbm.at[pl.ds(base + c * CHUNK, CHUNK)], osems[r])
        for slot in range(NBUF):
            pltpu.make_async_copy(
                acc_v.at[slot], out_hbm.at[pl.ds(base, CHUNK)],
                osems[slot]).wait()

    return k(table, idx)


_BLK = 400


def _dotT(x, w):
    return lax.dot_general(x, w, (((1,), (1,)), ((), ())),
                           preferred_element_type=jnp.float32)


def _tc_linear(h, agg, w_self, w_neigh, b):
    def body(h_ref, a_ref, ws_ref, wn_ref, b_ref, o_ref):
        acc = _dotT(h_ref[...], ws_ref[...])
        acc = acc + _dotT(a_ref[...] * (1.0 / DG), wn_ref[...])
        o_ref[...] = jnp.maximum(acc + b_ref[...], 0.0)

    return pl.pallas_call(
        body,
        grid=(NN // _BLK,),
        in_specs=[
            pl.BlockSpec((_BLK, DD), lambda i: (i, 0)),
            pl.BlockSpec((_BLK, DD), lambda i: (i, 0)),
            pl.BlockSpec((DD, DD), lambda i: (0, 0)),
            pl.BlockSpec((DD, DD), lambda i: (0, 0)),
            pl.BlockSpec((1, DD), lambda i: (0, 0)),
        ],
        out_specs=pl.BlockSpec((_BLK, DD), lambda i: (i, 0)),
        out_shape=jax.ShapeDtypeStruct((NN, DD), jnp.float32),
    )(h, agg, w_self, w_neigh, b)


def kernel(x, adj_lists, W1, b1, W2, b2):
    idx = adj_lists.astype(jnp.int32).reshape(-1)
    idx = jnp.pad(idx, (0, (NPAD - NN) * DG))
    h = x

    for W, b in ((W1, b1), (W2, b2)):
        agg = _sc_gather_sum(h, idx)
        h = _tc_linear(h, agg, W[:, :DD], W[:, DD:], b.reshape(1, DD))
    return h


# speedup vs baseline: 1.1326x; 1.1326x over previous
import functools

import jax
import jax.numpy as jnp
from jax import lax
from jax.experimental import pallas as pl
from jax.experimental.pallas import tpu as pltpu
from jax.experimental.pallas import tpu_sc as plsc

NN = 10000
DG = 16
DD = 256
NW = 32
NPAD = 10240
PER_W = NPAD // NW
CHUNK = 8
NCHUNK = PER_W // CHUNK
NBUF = 3
LANES = 16


def _sc_gather_sum(table, idx):
    mesh = plsc.VectorSubcoreMesh(core_axis_name="c", subcore_axis_name="s")

    @functools.partial(
        pl.kernel,
        mesh=mesh,
        out_type=jax.ShapeDtypeStruct((NPAD, DD), jnp.float32),
        scratch_types=(
            [pltpu.VMEM((PER_W * DG,), jnp.int32),
             pltpu.VMEM((NBUF, CHUNK * DG, DD), jnp.float32),
             pltpu.VMEM((NBUF, CHUNK, DD), jnp.float32)]
            + [pltpu.SemaphoreType.DMA] * (2 * NBUF)
        ),
    )
    def k(table_hbm, idx_hbm, out_hbm, idx_v, rows_v, acc_v, *sems):
        wid = lax.axis_index("s") * 2 + lax.axis_index("c")
        base = wid * PER_W
        pltpu.sync_copy(idx_hbm.at[pl.ds(base * DG, PER_W * DG)], idx_v)

        gsems = sems[:NBUF]
        osems = sems[NBUF:]

        def issue_gather(c, slot):
            return pltpu.async_copy(
                table_hbm.at[idx_v.at[pl.ds(c * CHUNK * DG, CHUNK * DG)]],
                rows_v.at[slot], gsems[slot])

        for slot in range(NBUF):
            issue_gather(slot, slot)

        def wait_gather(slot):
            pltpu.make_async_copy(
                table_hbm.at[pl.ds(0, CHUNK * DG)],
                rows_v.at[slot], gsems[slot]).wait()

        def reduce_slab(slot):
            def node_body(n, _):
                r0 = n * DG
                for d in range(DD // LANES):
                    sl = pl.ds(d * LANES, LANES)
                    vals = [rows_v[slot, r0 + j, sl] for j in range(DG)]
                    while len(vals) > 1:
                        vals = [vals[i] + vals[i + 1]
                                for i in range(0, len(vals), 2)]
                    acc_v[slot, n, sl] = vals[0]
                return 0
            lax.fori_loop(0, CHUNK, node_body, 0)

        def group_body(t, _):
            for slot in range(NBUF):
                c = NBUF * t + slot
                wait_gather(slot)

                @pl.when(t > 0)
                def _():
                    pltpu.make_async_copy(
                        acc_v.at[slot],
                        out_hbm.at[pl.ds(base, CHUNK)], osems[slot]).wait()

                reduce_slab(slot)

                @pl.when(c + NBUF < NCHUNK)
                def _():
                    issue_gather(c + NBUF, slot)

                pltpu.async_copy(
                    acc_v.at[slot],
                    out_hbm.at[pl.ds(base + c * CHUNK, CHUNK)], osems[slot])
            return 0

        ngroups = NCHUNK // NBUF
        lax.fori_loop(0, ngroups, group_body, 0)
        for r in range(NCHUNK % NBUF):
            c = ngroups * NBUF + r
            wait_gather(r)
            pltpu.make_async_copy(
                acc_v.at[r], out_hbm.at[pl.ds(base, CHUNK)],
                osems[r]).wait()
            reduce_slab(r)
            pltpu.async_copy(
                acc_v.at[r],
                out_hbm.at[pl.ds(base + c * CHUNK, CHUNK)], osems[r])
        for slot in range(NBUF):
            pltpu.make_async_copy(
                acc_v.at[slot], out_hbm.at[pl.ds(base, CHUNK)],
                osems[slot]).wait()

    return k(table, idx)


def _tc_linear(h, agg, w_self, w_neigh, b):
    blk = 512

    def body(h_ref, a_ref, ws_ref, wn_ref, b_ref, o_ref):
        acc = jnp.dot(h_ref[...], ws_ref[...],
                      preferred_element_type=jnp.float32)
        acc = acc + jnp.dot(a_ref[...], wn_ref[...],
                            preferred_element_type=jnp.float32)
        o_ref[...] = jnp.maximum(acc + b_ref[...], 0.0)

    return pl.pallas_call(
        body,
        grid=(NPAD // blk,),
        in_specs=[
            pl.BlockSpec((blk, DD), lambda i: (i, 0)),
            pl.BlockSpec((blk, DD), lambda i: (i, 0)),
            pl.BlockSpec((DD, DD), lambda i: (0, 0)),
            pl.BlockSpec((DD, DD), lambda i: (0, 0)),
            pl.BlockSpec((1, DD), lambda i: (0, 0)),
        ],
        out_specs=pl.BlockSpec((blk, DD), lambda i: (i, 0)),
        out_shape=jax.ShapeDtypeStruct((NPAD, DD), jnp.float32),
    )(h, agg, w_self, w_neigh, b)


def kernel(x, adj_lists, W1, b1, W2, b2):
    idx = adj_lists.astype(jnp.int32).reshape(-1)
    idx = jnp.pad(idx, (0, (NPAD - NN) * DG))
    h = jnp.pad(x, ((0, NPAD - NN), (0, 0)))

    for W, b in ((W1, b1), (W2, b2)):
        wt = W.T
        agg = _sc_gather_sum(h, idx)
        h = _tc_linear(h, agg, wt[:DD], wt[DD:] * (1.0 / DG),
                       b.reshape(1, DD))
    return h[:NN]
